# baseline (speedup 1.0000x reference)
import jax
import jax.numpy as jnp
from jax import lax
from jax.experimental import pallas as pl
from jax.experimental.pallas import tpu as pltpu
from jax.experimental.pallas import tpu_sc as plsc

NUM_CLASSES = 1000000
EMBED = 64
BATCH = 4096
SEQ = 200

NC = 2
NS = 16
NW = NC * NS
ROWS_PER_W = BATCH // NW

RB = 128
NBLK = (NUM_CLASSES + RB - 1) // RB
PAD_ROWS = NBLK * RB
ITERS_1 = (NBLK + NW - 1) // NW


NI = 4


def _tbody(tabT, tabR, in2, out2, si0, si1, si2, si3, so0, so1):
    wid = lax.axis_index("s") * NC + lax.axis_index("c")
    si = (si0, si1, si2, si3)
    so = (so0, so1)

    def start_in(blk, p):
        pltpu.make_async_copy(
            tabT.at[:, pl.ds(blk * RB, RB)], in2.at[p], si[p]
        ).start()

    def wait_in(p):
        pltpu.make_async_copy(
            tabT.at[:, pl.ds(0, RB)], in2.at[p], si[p]
        ).wait()

    def start_out(blk, p):
        pltpu.make_async_copy(
            out2.at[p], tabR.at[pl.ds(blk * RB, RB)], so[p]
        ).start()

    def wait_out(p):
        pltpu.make_async_copy(
            out2.at[p], tabR.at[pl.ds(0, RB)], so[p]
        ).wait()

    eye = lax.iota(jnp.int32, 16)

    def transpose_blk2(p, q):
        @plsc.parallel_loop(0, EMBED, step=1, unroll=4)
        def _(e):
            row = jnp.full((16,), e, jnp.int32)
            for cb in range(RB // 16):
                v = in2.at[p][e, pl.ds(cb * 16, 16)]
                plsc.store_scatter(out2.at[q], [eye + (cb * 16), row], v)

    for a in range(NI - 1):
        start_in(wid + a * NW, a)

    def step(i, carry):
        for j in range(NI):
            blk = wid + (NI * i + j) * NW
            p = j
            q = j % 2
            nxt = blk + (NI - 1) * NW

            @pl.when(blk < NBLK)
            def _():
                @pl.when(nxt < NBLK)
                def _():
                    start_in(nxt, (j + NI - 1) % NI)

                wait_in(p)

                @pl.when(blk >= 2 * NW)
                def _():
                    wait_out(q)

                transpose_blk2(p, q)
                start_out(blk, q)
        return carry

    lax.fori_loop(0, (ITERS_1 + NI - 1) // NI, step, 0)
    wait_out(0)
    wait_out(1)


CHUNKS = ((0, 128), (128, 72))
NG = 2
NO = 2


def _gbody(x_hbm, tabR, out_hbm, idx_all, rows, outb, sg0, sg1, so0, so1):
    wid = lax.axis_index("s") * NC + lax.axis_index("c")
    row0 = wid * ROWS_PER_W
    sg = (sg0, sg1)
    so = (so0, so1)

    pltpu.sync_copy(x_hbm.at[pl.ds(row0 * SEQ, ROWS_PER_W * SEQ)], idx_all)

    def start_gather(r, p):
        base = r * SEQ
        for off, n in CHUNKS:
            pltpu.make_async_copy(
                tabR.at[idx_all.at[pl.ds(base + off, n)]],
                rows.at[p].at[pl.ds(off, n)],
                sg[p],
            ).start()

    def wait_gather(p):
        for off, n in CHUNKS:
            pltpu.make_async_copy(
                tabR.at[pl.ds(0, n)],
                rows.at[p].at[pl.ds(off, n)],
                sg[p],
            ).wait()

    eye = lax.iota(jnp.int32, 16)

    def transpose(p, q):
        @plsc.parallel_loop(0, SEQ, step=1, unroll=4)
        def _(l):
            col = jnp.full((16,), l, jnp.int32)
            for eb in range(EMBED // 16):
                v = rows.at[p][l, pl.ds(eb * 16, 16)]
                plsc.store_scatter(outb.at[q], [eye + (eb * 16), col], v)

    def start_store(r, q):
        pltpu.make_async_copy(outb.at[q], out_hbm.at[row0 + r], so[q]).start()

    def wait_store(q):
        pltpu.make_async_copy(outb.at[q], out_hbm.at[row0], so[q]).wait()

    start_gather(0, 0)
    start_gather(1, 1)

    def step(k, carry):
        for j in range(2):
            r = 2 * k + j
            p = j
            q = j

            wait_gather(p)

            @pl.when(r >= NO)
            def _():
                wait_store(q)

            transpose(p, q)

            @pl.when(r + 2 < ROWS_PER_W)
            def _():
                start_gather(r + 2, p)

            start_store(r, q)
        return carry

    lax.fori_loop(0, ROWS_PER_W // 2, step, 0)
    wait_store(0)
    wait_store(1)


def _mesh():
    return plsc.VectorSubcoreMesh(
        core_axis_name="c", subcore_axis_name="s", num_cores=NC, num_subcores=NS
    )


_PARAMS = pltpu.CompilerParams(
    use_tc_tiling_on_sc=True, needs_layout_passes=False
)
_PARAMS_LINEAR = pltpu.CompilerParams(
    use_tc_tiling_on_sc=False, needs_layout_passes=False
)


_DEBUG_XLA_GATHER = False


@jax.jit
def _run(x, embed_table):
    t = pl.kernel(
        _tbody,
        out_type=jax.ShapeDtypeStruct((PAD_ROWS, 128), jnp.float32),
        mesh=_mesh(),
        scratch_types=[
            pltpu.VMEM((NI, EMBED, RB), jnp.float32),
            pltpu.VMEM((2, RB, 128), jnp.float32),
            pltpu.SemaphoreType.DMA,
            pltpu.SemaphoreType.DMA,
            pltpu.SemaphoreType.DMA,
            pltpu.SemaphoreType.DMA,
            pltpu.SemaphoreType.DMA,
            pltpu.SemaphoreType.DMA,
        ],
        compiler_params=_PARAMS,
    )
    tabR = t(embed_table.T)
    if _DEBUG_XLA_GATHER:
        emb = jnp.take(tabR[:NUM_CLASSES, :EMBED], x, axis=0)
        return jnp.transpose(emb, (0, 2, 1))
    g = pl.kernel(
        _gbody,
        out_type=jax.ShapeDtypeStruct((BATCH, EMBED, SEQ), jnp.float32),
        mesh=_mesh(),
        scratch_types=[
            pltpu.VMEM((ROWS_PER_W * SEQ,), jnp.int32),
            pltpu.VMEM((NG, SEQ, 128), jnp.float32),
            pltpu.VMEM((NO, EMBED, SEQ), jnp.float32),
            pltpu.SemaphoreType.DMA,
            pltpu.SemaphoreType.DMA,
            pltpu.SemaphoreType.DMA,
            pltpu.SemaphoreType.DMA,
        ],
        compiler_params=_PARAMS,
    )
    return g(x.reshape(-1), tabR)


def kernel(x, embed_table):
    return _run(x, embed_table)


# speedup vs baseline: 1.2257x; 1.2257x over previous
import jax
import jax.numpy as jnp
from jax import lax
from jax.experimental import pallas as pl
from jax.experimental.pallas import tpu as pltpu
from jax.experimental.pallas import tpu_sc as plsc

NUM_CLASSES = 1000000
EMBED = 64
BATCH = 4096
SEQ = 200

NC = 2
NS = 16
NW = NC * NS
ROWS_PER_W = BATCH // NW

CHUNKS = ((0, 128), (128, 72))
LSPLIT = 128


def _gbody(x_hbm, tabR, out_hbm, idx_all, rows, outa, outb, sg0, sg1, so0, so1):
    wid = lax.axis_index("s") * NC + lax.axis_index("c")
    row0 = wid * ROWS_PER_W
    sg = (sg0, sg1)
    so = (so0, so1)

    pltpu.sync_copy(x_hbm.at[pl.ds(row0 * SEQ, ROWS_PER_W * SEQ)], idx_all)

    def start_gather(r, p):
        base = r * SEQ
        for off, n in CHUNKS:
            pltpu.make_async_copy(
                tabR.at[idx_all.at[pl.ds(base + off, n)]],
                rows.at[p].at[pl.ds(off, n)],
                sg[p],
            ).start()

    def wait_gather(p):
        for off, n in CHUNKS:
            pltpu.make_async_copy(
                tabR.at[pl.ds(0, n)],
                rows.at[p].at[pl.ds(off, n)],
                sg[p],
            ).wait()

    eye = lax.iota(jnp.int32, 16)

    def transpose(p, q):
        @plsc.parallel_loop(0, LSPLIT, step=1, unroll=4)
        def _(l):
            col = jnp.full((16,), l, jnp.int32)
            for eb in range(EMBED // 16):
                v = rows.at[p][l, pl.ds(eb * 16, 16)]
                plsc.store_scatter(outa.at[q], [eye + (eb * 16), col], v)

        @plsc.parallel_loop(LSPLIT, SEQ, step=1, unroll=4)
        def _(l):
            col = jnp.full((16,), l - LSPLIT, jnp.int32)
            for eb in range(EMBED // 16):
                v = rows.at[p][l, pl.ds(eb * 16, 16)]
                plsc.store_scatter(outb.at[q], [eye + (eb * 16), col], v)

    def start_store(r, q):
        b = row0 + r
        pltpu.make_async_copy(
            outa.at[q], out_hbm.at[b, :, pl.ds(0, LSPLIT)], so[q]
        ).start()
        pltpu.make_async_copy(
            outb.at[q],
            out_hbm.at[b, :, pl.ds(LSPLIT, SEQ - LSPLIT)],
            so[q],
        ).start()

    def wait_store(q):
        pltpu.make_async_copy(
            outa.at[q], out_hbm.at[row0, :, pl.ds(0, LSPLIT)], so[q]
        ).wait()
        pltpu.make_async_copy(
            outb.at[q],
            out_hbm.at[row0, :, pl.ds(LSPLIT, SEQ - LSPLIT)],
            so[q],
        ).wait()

    start_gather(0, 0)
    start_gather(1, 1)

    def step(k, carry):
        for j in range(2):
            r = 2 * k + j
            p = j
            q = j

            wait_gather(p)

            @pl.when(r >= 2)
            def _():
                wait_store(q)

            transpose(p, q)

            @pl.when(r + 2 < ROWS_PER_W)
            def _():
                start_gather(r + 2, p)

            start_store(r, q)
        return carry

    lax.fori_loop(0, ROWS_PER_W // 2, step, 0)
    wait_store(0)
    wait_store(1)


@jax.jit
def _run(x, embed_table):
    tabR = jnp.pad(embed_table, ((0, 0), (0, 128 - EMBED)))
    g = pl.kernel(
        _gbody,
        out_type=jax.ShapeDtypeStruct((BATCH, EMBED, SEQ), jnp.float32),
        mesh=plsc.VectorSubcoreMesh(
            core_axis_name="c", subcore_axis_name="s",
            num_cores=NC, num_subcores=NS,
        ),
        scratch_types=[
            pltpu.VMEM((ROWS_PER_W * SEQ,), jnp.int32),
            pltpu.VMEM((2, SEQ, 128), jnp.float32),
            pltpu.VMEM((2, EMBED, 128), jnp.float32),
            pltpu.VMEM((2, EMBED, SEQ - LSPLIT), jnp.float32),
            pltpu.SemaphoreType.DMA,
            pltpu.SemaphoreType.DMA,
            pltpu.SemaphoreType.DMA,
            pltpu.SemaphoreType.DMA,
        ],
        compiler_params=pltpu.CompilerParams(
            use_tc_tiling_on_sc=True, needs_layout_passes=False
        ),
    )
    return g(x.reshape(-1), tabR)


def kernel(x, embed_table):
    return _run(x, embed_table)
